# baseline (speedup 1.0000x reference)
import functools

import jax
import jax.numpy as jnp
from jax import lax
from jax.experimental import pallas as pl
from jax.experimental.pallas import tpu as pltpu
from jax.experimental.pallas import tpu_sc as plsc

N = 10000
E = 320000
D = 128
R = 8
NC = 2
NS = 16
NW = NC * NS
CH = 128
NCHUNK = 79
EPW = NCHUNK * CH
E_PAD = NW * EPW
DACC = NS * 5008
OROWS = NS * 632
SKB0 = 119
SKB1 = 39
SKA0 = 105
SKA1 = 53
NB = 10
BR = N // NB

_f32 = jnp.float32
_i32 = jnp.int32


def _tc1_body(x_ref, w_ref, asr_ref, adr_ref, xl_ref, as_ref, ad_ref, ws_ref):
    xl = jnp.dot(x_ref[...], w_ref[...], preferred_element_type=_f32)
    xl_ref[...] = xl
    a_cols = []
    d_cols = []
    for r in range(R):
        seg = xl[:, D * (r + 1):D * (r + 2)]
        a_cols.append(jnp.sum(seg * asr_ref[r, :][None, :], axis=1, keepdims=True))
        d_cols.append(jnp.sum(seg * adr_ref[r, :][None, :], axis=1, keepdims=True))
    a = jnp.concatenate(a_cols, axis=1)
    d = jnp.concatenate(d_cols, axis=1)
    as_ref[...] = a
    ad_ref[...] = d
    v = a + d
    v = jnp.where(v >= 0.0, v, 0.2 * v)
    ws_ref[...] = jnp.exp(v)


def _tc2_body(xl_ref, ws_ref, d0_ref, d1_ref, bs_ref, b_ref, den_ref, outd_ref):
    ws = ws_ref[...]
    den = ws + d0_ref[...] + d1_ref[...]
    den_ref[...] = den
    xl = xl_ref[...]
    bsum = bs_ref[...] + jnp.sum(b_ref[...], axis=0)
    acc = xl[:, 0:D] + bsum[None, :]
    cs = ws / (den + 1e-16)
    for r in range(R):
        acc = acc + cs[:, r:r + 1] * xl[:, D * (r + 1):D * (r + 2)]
    outd_ref[...] = acc


def _tc3_body(a_ref, b_ref, c_ref, o_ref):
    o_ref[...] = a_ref[...] + b_ref[...] + c_ref[...]


_MESH = plsc.VectorSubcoreMesh(core_axis_name="c", subcore_axis_name="s")


@functools.partial(
    pl.kernel,
    out_type=[
        jax.ShapeDtypeStruct((E_PAD,), _f32),
        jax.ShapeDtypeStruct((R * N,), _f32),
        jax.ShapeDtypeStruct((R * N,), _f32),
    ],
    mesh=_MESH,
    scratch_types=[
        pltpu.VMEM((3 * CH,), _i32), pltpu.VMEM((3 * CH,), _i32),
        pltpu.VMEM((CH,), _i32), pltpu.VMEM((CH,), _i32),
        pltpu.VMEM((CH,), _i32), pltpu.VMEM((CH,), _i32),
        pltpu.VMEM((CH,), _i32), pltpu.VMEM((CH,), _i32),
        pltpu.VMEM((CH,), _f32), pltpu.VMEM((CH,), _f32),
        pltpu.VMEM((CH,), _f32), pltpu.VMEM((CH,), _f32),
        pltpu.VMEM((CH,), _f32), pltpu.VMEM((CH,), _f32),
        pltpu.VMEM((5008,), _f32),
        pltpu.VMEM_SHARED((DACC,), _f32),
        pltpu.SemaphoreType.DMA, pltpu.SemaphoreType.DMA,
        pltpu.SemaphoreType.DMA, pltpu.SemaphoreType.DMA,
    ],
)
def _sc_pass_a(edata, asf, adf, w_out, dp0, dp1,
               eb0, eb1, ia0, ia1, ibg0, ibg1, idn0, idn1, ga0, ga1,
               gb0, gb1, wv0, wv1, zbuf, dacc,
               se0, se1, sg0, sg1):
    c = lax.axis_index("c")
    s = lax.axis_index("s")
    my_k = jnp.where(c == 0, SKA0, SKA1)
    chunk0 = c * (NS * SKA0) + s * my_k
    B0 = (eb0, ia0, ibg0, idn0, ga0, gb0, wv0, se0, sg0)
    B1 = (eb1, ia1, ibg1, idn1, ga1, gb1, wv1, se1, sg1)

    def load_e(ci, B):
        eb, se = B[0], B[7]
        erow = (chunk0 + ci) * (3 * CH)
        pltpu.async_copy(edata.at[pl.ds(erow, 3 * CH)], eb, se)

    def wait_e(B):
        eb, se = B[0], B[7]
        pltpu.make_async_copy(edata.at[pl.ds(0, 3 * CH)], eb, se).wait()

    def idx_gather(B):
        eb, ia, ibg, idn, ga, gb, sg = B[0], B[1], B[2], B[3], B[4], B[5], B[8]
        for j in range(8):
            jl = pl.ds(j * 16, 16)
            s16 = eb[jl]
            d16 = eb[pl.ds(CH + j * 16, 16)]
            t16 = eb[pl.ds(2 * CH + j * 16, 16)]
            ia[jl] = s16 * R + t16
            dcl = jnp.minimum(d16, N - 1)
            ibg[jl] = dcl * R + t16
            idn[jl] = d16 * R + t16
        pltpu.async_copy(asf.at[ia], ga, sg)
        pltpu.async_copy(adf.at[ibg], gb, sg)

    def compute_out(ci, B):
        ia, ibg, idn, ga, gb, wv, sg = B[1], B[2], B[3], B[4], B[5], B[6], B[8]
        pltpu.make_async_copy(asf.at[ia], ga, sg).wait()
        pltpu.make_async_copy(adf.at[ibg], gb, sg).wait()
        for j in range(8):
            jl = pl.ds(j * 16, 16)
            v = ga[jl] + gb[jl]
            v = jnp.where(v >= 0.0, v, 0.2 * v)
            wv[jl] = jnp.exp(v)
        pltpu.sync_copy(wv, w_out.at[pl.ds((chunk0 + ci) * CH, CH)])
        pltpu.sync_copy(wv, dacc.at[idn], add=True)

    def zbody(i, _):
        zbuf[pl.ds(i * 16, 16)] = jnp.zeros((16,), _f32)
        return 0
    lax.fori_loop(0, 5008 // 16, zbody, 0)
    pltpu.sync_copy(zbuf, dacc.at[pl.ds(s * 5008, 5008)])
    plsc.subcore_barrier()

    load_e(0, B0)
    load_e(1, B1)
    wait_e(B0)
    idx_gather(B0)

    npairs = (my_k - 1) // 2

    def pair(k, _):
        c0 = 2 * k
        load_e(c0 + 2, B0)
        wait_e(B1)
        idx_gather(B1)
        compute_out(c0, B0)

        @pl.when(k < npairs - 1)
        def _():
            load_e(c0 + 3, B1)
        wait_e(B0)
        idx_gather(B0)
        compute_out(c0 + 1, B1)
        return 0
    lax.fori_loop(0, npairs, pair, 0)

    compute_out(my_k - 1, B0)
    plsc.subcore_barrier()

    pltpu.sync_copy(dacc.at[pl.ds(s * 5000, 5000)], zbuf.at[pl.ds(0, 5000)])

    @pl.when(c == 0)
    def _():
        pltpu.sync_copy(zbuf.at[pl.ds(0, 5000)], dp0.at[pl.ds(s * 5000, 5000)])

    @pl.when(c == 1)
    def _():
        pltpu.sync_copy(zbuf.at[pl.ds(0, 5000)], dp1.at[pl.ds(s * 5000, 5000)])


@functools.partial(
    pl.kernel,
    out_type=[
        jax.ShapeDtypeStruct((OROWS, D), _f32),
        jax.ShapeDtypeStruct((OROWS, D), _f32),
    ],
    mesh=_MESH,
    scratch_types=[
        pltpu.VMEM((3 * CH,), _i32), pltpu.VMEM((3 * CH,), _i32),
        pltpu.VMEM((CH,), _f32), pltpu.VMEM((CH,), _f32),
        pltpu.VMEM((CH,), _i32), pltpu.VMEM((CH,), _i32),
        pltpu.VMEM((CH,), _i32), pltpu.VMEM((CH,), _i32),
        pltpu.VMEM((CH,), _i32), pltpu.VMEM((CH,), _i32),
        pltpu.VMEM((CH,), _f32), pltpu.VMEM((CH,), _f32),
        pltpu.VMEM((CH,), _f32), pltpu.VMEM((CH,), _f32),
        pltpu.VMEM((CH, D), _f32), pltpu.VMEM((CH, D), _f32),
        pltpu.VMEM_SHARED((OROWS, D), _f32),
        pltpu.SemaphoreType.DMA, pltpu.SemaphoreType.DMA,
        pltpu.SemaphoreType.DMA, pltpu.SemaphoreType.DMA,
        pltpu.SemaphoreType.DMA, pltpu.SemaphoreType.DMA,
    ],
)
def _sc_pass_b(edata, wf, denf, xlt, op0, op1,
               eb0, eb1, wv0, wv1, ix0, ix1, idn0, idn1, io0, io1,
               gd0, gd1, cf0, cf1, rows0, rows1, oacc,
               se0, se1, sg0, sg1, sw0, sw1):
    c = lax.axis_index("c")
    s = lax.axis_index("s")
    my_k = jnp.where(c == 0, SKB0, SKB1)
    chunk0 = c * (NS * SKB0) + s * my_k
    B0 = (eb0, wv0, ix0, idn0, io0, gd0, cf0, rows0, se0, sg0, sw0)
    B1 = (eb1, wv1, ix1, idn1, io1, gd1, cf1, rows1, se1, sg1, sw1)

    def load_e(ci, B):
        eb, se = B[0], B[8]
        erow = (chunk0 + ci) * (3 * CH)
        pltpu.async_copy(edata.at[pl.ds(erow, 3 * CH)], eb, se)

    def load_w(ci, B):
        wv, sw = B[1], B[10]
        pltpu.async_copy(wf.at[pl.ds((chunk0 + ci) * CH, CH)], wv, sw)

    def wait_e(B):
        eb, se = B[0], B[8]
        pltpu.make_async_copy(edata.at[pl.ds(0, 3 * CH)], eb, se).wait()

    def idx_gather(B):
        eb, ix, idn, io, gd, rows, sg = B[0], B[2], B[3], B[4], B[5], B[7], B[9]
        for j in range(8):
            jl = pl.ds(j * 16, 16)
            s16 = eb[jl]
            d16 = eb[pl.ds(CH + j * 16, 16)]
            t16 = eb[pl.ds(2 * CH + j * 16, 16)]
            ix[jl] = s16 * 9 + t16 + 1
            dcl = jnp.minimum(d16, N - 1)
            idn[jl] = dcl * R + t16
            io[jl] = d16
        pltpu.async_copy(denf.at[idn], gd, sg)
        pltpu.async_copy(xlt.at[ix], rows, sg)

    def compute_scat(B):
        ix, idn, io, gd, cf, rows, wv, sg = (
            B[2], B[3], B[4], B[5], B[6], B[7], B[1], B[9])
        sw = B[10]
        pltpu.make_async_copy(denf.at[idn], gd, sg).wait()
        pltpu.make_async_copy(xlt.at[ix], rows, sg).wait()
        pltpu.make_async_copy(wf.at[pl.ds(0, CH)], wv, sw).wait()
        for j in range(8):
            jl = pl.ds(j * 16, 16)
            cf[jl] = wv[jl] / (gd[jl] + 1e-16)
        def scale(g, _):
            cvec = cf[pl.ds(g * 16, 16)]
            for k in range(16):
                cv = cvec[k]
                row = g * 16 + k
                for j in range(D // 16):
                    rows[row, pl.ds(j * 16, 16)] = rows[row, pl.ds(j * 16, 16)] * cv
            return 0
        lax.fori_loop(0, CH // 16, scale, 0)
        pltpu.sync_copy(rows, oacc.at[io], add=True)

    def zrows(i, _):
        for j in range(D // 16):
            rows0[i, pl.ds(j * 16, 16)] = jnp.zeros((16,), _f32)
        return 0
    lax.fori_loop(0, CH, zrows, 0)
    r0 = s * 632
    for k in range(4):
        pltpu.sync_copy(rows0, oacc.at[pl.ds(r0 + k * CH, CH)])
    pltpu.sync_copy(rows0.at[pl.ds(0, 120)], oacc.at[pl.ds(r0 + 4 * CH, 120)])
    plsc.subcore_barrier()

    load_e(0, B0)
    load_e(1, B1)
    wait_e(B0)
    idx_gather(B0)
    load_w(0, B0)

    npairs = (my_k - 1) // 2

    def pair(k, _):
        c0 = 2 * k
        wait_e(B1)
        idx_gather(B1)
        load_w(c0 + 1, B1)
        compute_scat(B0)
        load_e(c0 + 2, B0)
        wait_e(B0)
        idx_gather(B0)
        load_w(c0 + 2, B0)

        @pl.when(k < npairs - 1)
        def _():
            load_e(c0 + 3, B1)
        compute_scat(B1)
        return 0
    lax.fori_loop(0, npairs, pair, 0)

    compute_scat(B0)
    plsc.subcore_barrier()

    for k in range(5):
        nr = CH if k < 4 else 120
        pltpu.sync_copy(oacc.at[pl.ds(r0 + k * CH, nr)], rows0.at[pl.ds(0, nr)])

        @pl.when(c == 0)
        def _():
            pltpu.sync_copy(rows0.at[pl.ds(0, nr)], op0.at[pl.ds(r0 + k * CH, nr)])

        @pl.when(c == 1)
        def _():
            pltpu.sync_copy(rows0.at[pl.ds(0, nr)], op1.at[pl.ds(r0 + k * CH, nr)])


def kernel(x, edge_index, edge_type, W_self, b_self, W, att_src, att_dst, b):
    wcat = jnp.concatenate(
        [W_self[:, None, :], jnp.transpose(W, (1, 0, 2))], axis=1
    ).reshape(D, 9 * D)

    pad = E_PAD - E
    srcp = jnp.concatenate([edge_index[0], jnp.zeros((pad,), _i32)])
    dstp = jnp.concatenate([edge_index[1], jnp.full((pad,), N, _i32)])
    typp = jnp.concatenate([edge_type, jnp.zeros((pad,), _i32)])
    edata = jnp.concatenate(
        [srcp.reshape(-1, CH)[:, None, :], dstp.reshape(-1, CH)[:, None, :],
         typp.reshape(-1, CH)[:, None, :]], axis=1).reshape(-1)

    xl, asn, adn, wsn = pl.pallas_call(
        _tc1_body,
        grid=(NB,),
        in_specs=[
            pl.BlockSpec((BR, D), lambda i: (i, 0)),
            pl.BlockSpec((D, 9 * D), lambda i: (0, 0)),
            pl.BlockSpec((R, D), lambda i: (0, 0)),
            pl.BlockSpec((R, D), lambda i: (0, 0)),
        ],
        out_specs=[
            pl.BlockSpec((BR, 9 * D), lambda i: (i, 0)),
            pl.BlockSpec((BR, R), lambda i: (i, 0)),
            pl.BlockSpec((BR, R), lambda i: (i, 0)),
            pl.BlockSpec((BR, R), lambda i: (i, 0)),
        ],
        out_shape=[
            jax.ShapeDtypeStruct((N, 9 * D), _f32),
            jax.ShapeDtypeStruct((N, R), _f32),
            jax.ShapeDtypeStruct((N, R), _f32),
            jax.ShapeDtypeStruct((N, R), _f32),
        ],
    )(x, wcat, att_src, att_dst)

    w_e, dp0, dp1 = _sc_pass_a(
        edata, asn.reshape(R * N), adn.reshape(R * N)
    )

    den, outd = pl.pallas_call(
        _tc2_body,
        grid=(NB,),
        in_specs=[
            pl.BlockSpec((BR, 9 * D), lambda i: (i, 0)),
            pl.BlockSpec((BR, R), lambda i: (i, 0)),
            pl.BlockSpec((BR, R), lambda i: (i, 0)),
            pl.BlockSpec((BR, R), lambda i: (i, 0)),
            pl.BlockSpec((D,), lambda i: (0,)),
            pl.BlockSpec((R, D), lambda i: (0, 0)),
        ],
        out_specs=[
            pl.BlockSpec((BR, R), lambda i: (i, 0)),
            pl.BlockSpec((BR, D), lambda i: (i, 0)),
        ],
        out_shape=[
            jax.ShapeDtypeStruct((N, R), _f32),
            jax.ShapeDtypeStruct((N, D), _f32),
        ],
    )(xl, wsn, dp0.reshape(N, R), dp1.reshape(N, R), b_self, b)

    op0, op1 = _sc_pass_b(
        edata, w_e, den.reshape(R * N), xl.reshape(9 * N, D)
    )

    out = pl.pallas_call(
        _tc3_body,
        grid=(NB,),
        in_specs=[
            pl.BlockSpec((BR, D), lambda i: (i, 0)),
            pl.BlockSpec((BR, D), lambda i: (i, 0)),
            pl.BlockSpec((BR, D), lambda i: (i, 0)),
        ],
        out_specs=pl.BlockSpec((BR, D), lambda i: (i, 0)),
        out_shape=jax.ShapeDtypeStruct((N, D), _f32),
    )(outd, op0, op1)
    return out


# speedup vs baseline: 1.0336x; 1.0336x over previous
import functools

import jax
import jax.numpy as jnp
from jax import lax
from jax.experimental import pallas as pl
from jax.experimental.pallas import tpu as pltpu
from jax.experimental.pallas import tpu_sc as plsc

N = 10000
E = 320000
D = 128
R = 8
NC = 2
NS = 16
NW = NC * NS
CH = 128
NCHUNK = 79
EPW = NCHUNK * CH
E_PAD = NW * EPW
DACC = NS * 5008
OROWS = NS * 632
SKB0 = 119
SKB1 = 39
SKA0 = 79
SKA1 = 79
NB = 10
BR = N // NB

_f32 = jnp.float32
_i32 = jnp.int32


def _tc1_body(x_ref, w_ref, asr_ref, adr_ref, xl_ref, as_ref, ad_ref, ws_ref):
    xl = jnp.dot(x_ref[...], w_ref[...], preferred_element_type=_f32)
    xl_ref[...] = xl
    a_cols = []
    d_cols = []
    for r in range(R):
        seg = xl[:, D * (r + 1):D * (r + 2)]
        a_cols.append(jnp.sum(seg * asr_ref[r, :][None, :], axis=1, keepdims=True))
        d_cols.append(jnp.sum(seg * adr_ref[r, :][None, :], axis=1, keepdims=True))
    a = jnp.concatenate(a_cols, axis=1)
    d = jnp.concatenate(d_cols, axis=1)
    as_ref[...] = a
    ad_ref[...] = d
    v = a + d
    v = jnp.where(v >= 0.0, v, 0.2 * v)
    ws_ref[...] = jnp.exp(v)


def _tc2_body(xl_ref, ws_ref, d0_ref, d1_ref, bs_ref, b_ref, den_ref, outd_ref):
    ws = ws_ref[...]
    den = ws + d0_ref[...] + d1_ref[...]
    den_ref[...] = den
    xl = xl_ref[...]
    bsum = bs_ref[...] + jnp.sum(b_ref[...], axis=0)
    acc = xl[:, 0:D] + bsum[None, :]
    cs = ws / (den + 1e-16)
    for r in range(R):
        acc = acc + cs[:, r:r + 1] * xl[:, D * (r + 1):D * (r + 2)]
    outd_ref[...] = acc


def _tc3_body(a_ref, b_ref, c_ref, o_ref):
    o_ref[...] = a_ref[...] + b_ref[...] + c_ref[...]


_MESH = plsc.VectorSubcoreMesh(core_axis_name="c", subcore_axis_name="s")


@functools.partial(
    pl.kernel,
    out_type=[
        jax.ShapeDtypeStruct((E_PAD,), _f32),
        jax.ShapeDtypeStruct((R * N,), _f32),
        jax.ShapeDtypeStruct((R * N,), _f32),
    ],
    mesh=_MESH,
    scratch_types=[
        pltpu.VMEM((3 * CH,), _i32), pltpu.VMEM((3 * CH,), _i32),
        pltpu.VMEM((CH,), _i32), pltpu.VMEM((CH,), _i32),
        pltpu.VMEM((CH,), _i32), pltpu.VMEM((CH,), _i32),
        pltpu.VMEM((CH,), _i32), pltpu.VMEM((CH,), _i32),
        pltpu.VMEM((CH,), _f32), pltpu.VMEM((CH,), _f32),
        pltpu.VMEM((CH,), _f32), pltpu.VMEM((CH,), _f32),
        pltpu.VMEM((CH,), _f32), pltpu.VMEM((CH,), _f32),
        pltpu.VMEM((5008,), _f32),
        pltpu.VMEM_SHARED((DACC,), _f32),
        pltpu.SemaphoreType.DMA, pltpu.SemaphoreType.DMA,
        pltpu.SemaphoreType.DMA, pltpu.SemaphoreType.DMA,
    ],
)
def _sc_pass_a(edata, asf, adf, w_out, dp0, dp1,
               eb0, eb1, ia0, ia1, ibg0, ibg1, idn0, idn1, ga0, ga1,
               gb0, gb1, wv0, wv1, zbuf, dacc,
               se0, se1, sg0, sg1):
    c = lax.axis_index("c")
    s = lax.axis_index("s")
    my_k = jnp.where(c == 0, SKA0, SKA1)
    chunk0 = c * (NS * SKA0) + s * my_k
    B0 = (eb0, ia0, ibg0, idn0, ga0, gb0, wv0, se0, sg0)
    B1 = (eb1, ia1, ibg1, idn1, ga1, gb1, wv1, se1, sg1)

    def load_e(ci, B):
        eb, se = B[0], B[7]
        erow = (chunk0 + ci) * (3 * CH)
        pltpu.async_copy(edata.at[pl.ds(erow, 3 * CH)], eb, se)

    def wait_e(B):
        eb, se = B[0], B[7]
        pltpu.make_async_copy(edata.at[pl.ds(0, 3 * CH)], eb, se).wait()

    def idx_gather(B):
        eb, ia, ibg, idn, ga, gb, sg = B[0], B[1], B[2], B[3], B[4], B[5], B[8]
        for j in range(8):
            jl = pl.ds(j * 16, 16)
            s16 = eb[jl]
            d16 = eb[pl.ds(CH + j * 16, 16)]
            t16 = eb[pl.ds(2 * CH + j * 16, 16)]
            ia[jl] = s16 * R + t16
            dcl = jnp.minimum(d16, N - 1)
            ibg[jl] = dcl * R + t16
            idn[jl] = d16 * R + t16
        pltpu.async_copy(asf.at[ia], ga, sg)
        pltpu.async_copy(adf.at[ibg], gb, sg)

    def compute_out(ci, B):
        ia, ibg, idn, ga, gb, wv, sg = B[1], B[2], B[3], B[4], B[5], B[6], B[8]
        pltpu.make_async_copy(asf.at[ia], ga, sg).wait()
        pltpu.make_async_copy(adf.at[ibg], gb, sg).wait()
        for j in range(8):
            jl = pl.ds(j * 16, 16)
            v = ga[jl] + gb[jl]
            v = jnp.where(v >= 0.0, v, 0.2 * v)
            wv[jl] = jnp.exp(v)
        pltpu.sync_copy(wv, w_out.at[pl.ds((chunk0 + ci) * CH, CH)])
        pltpu.sync_copy(wv, dacc.at[idn], add=True)

    def zbody(i, _):
        zbuf[pl.ds(i * 16, 16)] = jnp.zeros((16,), _f32)
        return 0
    lax.fori_loop(0, 5008 // 16, zbody, 0)
    pltpu.sync_copy(zbuf, dacc.at[pl.ds(s * 5008, 5008)])
    plsc.subcore_barrier()

    load_e(0, B0)
    load_e(1, B1)
    wait_e(B0)
    idx_gather(B0)

    npairs = (my_k - 1) // 2

    def pair(k, _):
        c0 = 2 * k
        load_e(c0 + 2, B0)
        wait_e(B1)
        idx_gather(B1)
        compute_out(c0, B0)

        @pl.when(k < npairs - 1)
        def _():
            load_e(c0 + 3, B1)
        wait_e(B0)
        idx_gather(B0)
        compute_out(c0 + 1, B1)
        return 0
    lax.fori_loop(0, npairs, pair, 0)

    compute_out(my_k - 1, B0)
    plsc.subcore_barrier()

    pltpu.sync_copy(dacc.at[pl.ds(s * 5000, 5000)], zbuf.at[pl.ds(0, 5000)])

    @pl.when(c == 0)
    def _():
        pltpu.sync_copy(zbuf.at[pl.ds(0, 5000)], dp0.at[pl.ds(s * 5000, 5000)])

    @pl.when(c == 1)
    def _():
        pltpu.sync_copy(zbuf.at[pl.ds(0, 5000)], dp1.at[pl.ds(s * 5000, 5000)])


@functools.partial(
    pl.kernel,
    out_type=[
        jax.ShapeDtypeStruct((OROWS, D), _f32),
        jax.ShapeDtypeStruct((OROWS, D), _f32),
    ],
    mesh=_MESH,
    scratch_types=[
        pltpu.VMEM((3 * CH,), _i32), pltpu.VMEM((3 * CH,), _i32),
        pltpu.VMEM((CH,), _f32), pltpu.VMEM((CH,), _f32),
        pltpu.VMEM((CH,), _i32), pltpu.VMEM((CH,), _i32),
        pltpu.VMEM((CH,), _i32), pltpu.VMEM((CH,), _i32),
        pltpu.VMEM((CH,), _i32), pltpu.VMEM((CH,), _i32),
        pltpu.VMEM((CH,), _f32), pltpu.VMEM((CH,), _f32),
        pltpu.VMEM((CH,), _f32), pltpu.VMEM((CH,), _f32),
        pltpu.VMEM((CH, D), _f32), pltpu.VMEM((CH, D), _f32),
        pltpu.VMEM_SHARED((OROWS, D), _f32),
        pltpu.SemaphoreType.DMA, pltpu.SemaphoreType.DMA,
        pltpu.SemaphoreType.DMA, pltpu.SemaphoreType.DMA,
        pltpu.SemaphoreType.DMA, pltpu.SemaphoreType.DMA,
    ],
)
def _sc_pass_b(edata, wf, denf, xlt, op0, op1,
               eb0, eb1, wv0, wv1, ix0, ix1, idn0, idn1, io0, io1,
               gd0, gd1, cf0, cf1, rows0, rows1, oacc,
               se0, se1, sg0, sg1, sw0, sw1):
    c = lax.axis_index("c")
    s = lax.axis_index("s")
    my_k = jnp.where(c == 0, SKB0, SKB1)
    chunk0 = c * (NS * SKB0) + s * my_k
    B0 = (eb0, wv0, ix0, idn0, io0, gd0, cf0, rows0, se0, sg0, sw0)
    B1 = (eb1, wv1, ix1, idn1, io1, gd1, cf1, rows1, se1, sg1, sw1)

    def load_e(ci, B):
        eb, se = B[0], B[8]
        erow = (chunk0 + ci) * (3 * CH)
        pltpu.async_copy(edata.at[pl.ds(erow, 3 * CH)], eb, se)

    def load_w(ci, B):
        wv, sw = B[1], B[10]
        pltpu.async_copy(wf.at[pl.ds((chunk0 + ci) * CH, CH)], wv, sw)

    def wait_e(B):
        eb, se = B[0], B[8]
        pltpu.make_async_copy(edata.at[pl.ds(0, 3 * CH)], eb, se).wait()

    def idx_gather(B):
        eb, ix, idn, io, gd, rows, sg = B[0], B[2], B[3], B[4], B[5], B[7], B[9]
        for j in range(8):
            jl = pl.ds(j * 16, 16)
            s16 = eb[jl]
            d16 = eb[pl.ds(CH + j * 16, 16)]
            t16 = eb[pl.ds(2 * CH + j * 16, 16)]
            ix[jl] = s16 * 9 + t16 + 1
            dcl = jnp.minimum(d16, N - 1)
            idn[jl] = dcl * R + t16
            io[jl] = d16
        pltpu.async_copy(denf.at[idn], gd, sg)
        pltpu.async_copy(xlt.at[ix], rows, sg)

    def compute_scat(B):
        ix, idn, io, gd, cf, rows, wv, sg = (
            B[2], B[3], B[4], B[5], B[6], B[7], B[1], B[9])
        sw = B[10]
        pltpu.make_async_copy(denf.at[idn], gd, sg).wait()
        pltpu.make_async_copy(xlt.at[ix], rows, sg).wait()
        pltpu.make_async_copy(wf.at[pl.ds(0, CH)], wv, sw).wait()
        for j in range(8):
            jl = pl.ds(j * 16, 16)
            cf[jl] = wv[jl] / (gd[jl] + 1e-16)
        def scale(g, _):
            cvec = cf[pl.ds(g * 16, 16)]
            for k in range(16):
                cv = cvec[k]
                row = g * 16 + k
                for j in range(D // 16):
                    rows[row, pl.ds(j * 16, 16)] = rows[row, pl.ds(j * 16, 16)] * cv
            return 0
        lax.fori_loop(0, CH // 16, scale, 0)
        pltpu.sync_copy(rows, oacc.at[io], add=True)

    def zrows(i, _):
        for j in range(D // 16):
            rows0[i, pl.ds(j * 16, 16)] = jnp.zeros((16,), _f32)
        return 0
    lax.fori_loop(0, CH, zrows, 0)
    r0 = s * 632
    for k in range(4):
        pltpu.sync_copy(rows0, oacc.at[pl.ds(r0 + k * CH, CH)])
    pltpu.sync_copy(rows0.at[pl.ds(0, 120)], oacc.at[pl.ds(r0 + 4 * CH, 120)])
    plsc.subcore_barrier()

    load_e(0, B0)
    load_e(1, B1)
    wait_e(B0)
    idx_gather(B0)
    load_w(0, B0)

    npairs = (my_k - 1) // 2

    def pair(k, _):
        c0 = 2 * k
        wait_e(B1)
        idx_gather(B1)
        load_w(c0 + 1, B1)
        compute_scat(B0)
        load_e(c0 + 2, B0)
        wait_e(B0)
        idx_gather(B0)
        load_w(c0 + 2, B0)

        @pl.when(k < npairs - 1)
        def _():
            load_e(c0 + 3, B1)
        compute_scat(B1)
        return 0
    lax.fori_loop(0, npairs, pair, 0)

    compute_scat(B0)
    plsc.subcore_barrier()

    for k in range(5):
        nr = CH if k < 4 else 120
        pltpu.sync_copy(oacc.at[pl.ds(r0 + k * CH, nr)], rows0.at[pl.ds(0, nr)])

        @pl.when(c == 0)
        def _():
            pltpu.sync_copy(rows0.at[pl.ds(0, nr)], op0.at[pl.ds(r0 + k * CH, nr)])

        @pl.when(c == 1)
        def _():
            pltpu.sync_copy(rows0.at[pl.ds(0, nr)], op1.at[pl.ds(r0 + k * CH, nr)])


def kernel(x, edge_index, edge_type, W_self, b_self, W, att_src, att_dst, b):
    wcat = jnp.concatenate(
        [W_self[:, None, :], jnp.transpose(W, (1, 0, 2))], axis=1
    ).reshape(D, 9 * D)

    pad = E_PAD - E
    srcp = jnp.concatenate([edge_index[0], jnp.zeros((pad,), _i32)])
    dstp = jnp.concatenate([edge_index[1], jnp.full((pad,), N, _i32)])
    typp = jnp.concatenate([edge_type, jnp.zeros((pad,), _i32)])
    edata = jnp.concatenate(
        [srcp.reshape(-1, CH)[:, None, :], dstp.reshape(-1, CH)[:, None, :],
         typp.reshape(-1, CH)[:, None, :]], axis=1).reshape(-1)

    xl, asn, adn, wsn = pl.pallas_call(
        _tc1_body,
        grid=(NB,),
        in_specs=[
            pl.BlockSpec((BR, D), lambda i: (i, 0)),
            pl.BlockSpec((D, 9 * D), lambda i: (0, 0)),
            pl.BlockSpec((R, D), lambda i: (0, 0)),
            pl.BlockSpec((R, D), lambda i: (0, 0)),
        ],
        out_specs=[
            pl.BlockSpec((BR, 9 * D), lambda i: (i, 0)),
            pl.BlockSpec((BR, R), lambda i: (i, 0)),
            pl.BlockSpec((BR, R), lambda i: (i, 0)),
            pl.BlockSpec((BR, R), lambda i: (i, 0)),
        ],
        out_shape=[
            jax.ShapeDtypeStruct((N, 9 * D), _f32),
            jax.ShapeDtypeStruct((N, R), _f32),
            jax.ShapeDtypeStruct((N, R), _f32),
            jax.ShapeDtypeStruct((N, R), _f32),
        ],
    )(x, wcat, att_src, att_dst)

    w_e, dp0, dp1 = _sc_pass_a(
        edata, asn.reshape(R * N), adn.reshape(R * N)
    )

    den, outd = pl.pallas_call(
        _tc2_body,
        grid=(NB,),
        in_specs=[
            pl.BlockSpec((BR, 9 * D), lambda i: (i, 0)),
            pl.BlockSpec((BR, R), lambda i: (i, 0)),
            pl.BlockSpec((BR, R), lambda i: (i, 0)),
            pl.BlockSpec((BR, R), lambda i: (i, 0)),
            pl.BlockSpec((D,), lambda i: (0,)),
            pl.BlockSpec((R, D), lambda i: (0, 0)),
        ],
        out_specs=[
            pl.BlockSpec((BR, R), lambda i: (i, 0)),
            pl.BlockSpec((BR, D), lambda i: (i, 0)),
        ],
        out_shape=[
            jax.ShapeDtypeStruct((N, R), _f32),
            jax.ShapeDtypeStruct((N, D), _f32),
        ],
    )(xl, wsn, dp0.reshape(N, R), dp1.reshape(N, R), b_self, b)

    op0, op1 = _sc_pass_b(
        edata, w_e, den.reshape(R * N), xl.reshape(9 * N, D)
    )

    out = pl.pallas_call(
        _tc3_body,
        grid=(NB,),
        in_specs=[
            pl.BlockSpec((BR, D), lambda i: (i, 0)),
            pl.BlockSpec((BR, D), lambda i: (i, 0)),
            pl.BlockSpec((BR, D), lambda i: (i, 0)),
        ],
        out_specs=pl.BlockSpec((BR, D), lambda i: (i, 0)),
        out_shape=jax.ShapeDtypeStruct((N, D), _f32),
    )(outd, op0, op1)
    return out
